# baseline (speedup 1.0000x reference)
import functools

import jax
import jax.numpy as jnp
from jax import lax
from jax.experimental import pallas as pl
from jax.experimental.pallas import tpu as pltpu
from jax.experimental.pallas import tpu_sc as plsc

NBUF = 3
VBLK = 2000


def _sc_gather(V, D, B):
    info = plsc.get_sparse_core_info()
    nc, ns = info.num_cores, info.num_subcores
    nw = nc * ns
    b_per_w = B // nw
    mesh = plsc.VectorSubcoreMesh(core_axis_name="c", subcore_axis_name="s")

    @functools.partial(
        pl.kernel,
        mesh=mesh,
        out_type=jax.ShapeDtypeStruct((B, D), jnp.float32),
        scratch_types=[
            pltpu.VMEM((b_per_w,), jnp.int32),
            pltpu.VMEM((b_per_w, D), jnp.float32),
            pltpu.SemaphoreType.DMA,
        ],
    )
    def gather_kernel(table_hbm, idx_hbm, out_hbm, idx_v, rows_v, sem):
        wid = lax.axis_index("s") * nc + lax.axis_index("c")
        base = wid * b_per_w
        pltpu.sync_copy(idx_hbm.at[pl.ds(base, b_per_w)], idx_v)
        for c in range(b_per_w // 16):
            vec = idx_v[pl.ds(c * 16, 16)]
            for l in range(16):
                r = c * 16 + l
                pltpu.async_copy(table_hbm.at[vec[l]], rows_v.at[r], sem).start()
        for r in range(b_per_w):
            pltpu.make_async_copy(table_hbm.at[0], rows_v.at[r], sem).wait()
        pltpu.sync_copy(rows_v, out_hbm.at[pl.ds(base, b_per_w)])

    return gather_kernel


def _mm_body(emb_ref, w_ref, out_hbm, slabs, sems):
    i = pl.program_id(0)
    nb = pl.num_programs(0)
    slot = lax.rem(i, NBUF)

    @pl.when(i >= NBUF)
    def _wait_prev():
        pltpu.make_async_copy(
            slabs.at[slot],
            out_hbm.at[pl.ds((i - NBUF) * VBLK, VBLK), :],
            sems.at[slot],
        ).wait()

    slabs[slot] = lax.dot_general(
        w_ref[...],
        emb_ref[...],
        dimension_numbers=(((1,), (1,)), ((), ())),
        preferred_element_type=jnp.float32,
    )
    pltpu.make_async_copy(
        slabs.at[slot],
        out_hbm.at[pl.ds(i * VBLK, VBLK), :],
        sems.at[slot],
    ).start()

    @pl.when(i == nb - 1)
    def _drain():
        for s in range(NBUF):
            step = nb - NBUF + s
            sl = lax.rem(jnp.int32(step), NBUF)
            pltpu.make_async_copy(
                slabs.at[sl],
                out_hbm.at[pl.ds(step * VBLK, VBLK), :],
                sems.at[sl],
            ).wait()


def kernel(x, W):
    V, H = W.shape
    (B,) = x.shape
    idx = x.astype(jnp.int32)
    emb = _sc_gather(V, H, B)(W, idx)

    num_blocks = V // VBLK
    outT = pl.pallas_call(
        _mm_body,
        grid=(num_blocks,),
        in_specs=[
            pl.BlockSpec((B, H), lambda i: (0, 0)),
            pl.BlockSpec((VBLK, H), lambda i: (i, 0)),
        ],
        out_specs=pl.BlockSpec(memory_space=pltpu.MemorySpace.HBM),
        out_shape=jax.ShapeDtypeStruct((V, B), jnp.float32),
        scratch_shapes=[
            pltpu.VMEM((NBUF, VBLK, B), jnp.float32),
            pltpu.SemaphoreType.DMA((NBUF,)),
        ],
    )(emb, W)
    return outT.T


# speedup vs baseline: 1.0160x; 1.0160x over previous
import functools

import jax
import jax.numpy as jnp
from jax import lax
from jax.experimental import pallas as pl
from jax.experimental.pallas import tpu as pltpu
from jax.experimental.pallas import tpu_sc as plsc

NBUF = 2
VBLK = 5000


def _sc_gather(V, D, B):
    info = plsc.get_sparse_core_info()
    nc, ns = info.num_cores, info.num_subcores
    nw = nc * ns
    b_per_w = B // nw
    mesh = plsc.VectorSubcoreMesh(core_axis_name="c", subcore_axis_name="s")

    @functools.partial(
        pl.kernel,
        mesh=mesh,
        out_type=jax.ShapeDtypeStruct((B, D), jnp.float32),
        scratch_types=[
            pltpu.VMEM((b_per_w,), jnp.int32),
            pltpu.VMEM((b_per_w, D), jnp.float32),
            pltpu.SemaphoreType.DMA,
        ],
    )
    def gather_kernel(table_hbm, idx_hbm, out_hbm, idx_v, rows_v, sem):
        wid = lax.axis_index("s") * nc + lax.axis_index("c")
        base = wid * b_per_w
        pltpu.sync_copy(idx_hbm.at[pl.ds(base, b_per_w)], idx_v)
        for c in range(b_per_w // 16):
            vec = idx_v[pl.ds(c * 16, 16)]
            for l in range(16):
                r = c * 16 + l
                pltpu.async_copy(table_hbm.at[vec[l]], rows_v.at[r], sem).start()
        for r in range(b_per_w):
            pltpu.make_async_copy(table_hbm.at[0], rows_v.at[r], sem).wait()
        pltpu.sync_copy(rows_v, out_hbm.at[pl.ds(base, b_per_w)])

    return gather_kernel


def _mm_body(emb_ref, w_ref, out_hbm, slabs, sems):
    i = pl.program_id(0)
    nb = pl.num_programs(0)
    slot = lax.rem(i, NBUF)

    @pl.when(i >= NBUF)
    def _wait_prev():
        pltpu.make_async_copy(
            slabs.at[slot],
            out_hbm.at[pl.ds((i - NBUF) * VBLK, VBLK), :],
            sems.at[slot],
        ).wait()

    slabs[slot] = lax.dot_general(
        w_ref[...],
        emb_ref[...],
        dimension_numbers=(((1,), (1,)), ((), ())),
        preferred_element_type=jnp.float32,
    )
    pltpu.make_async_copy(
        slabs.at[slot],
        out_hbm.at[pl.ds(i * VBLK, VBLK), :],
        sems.at[slot],
    ).start()

    @pl.when(i == nb - 1)
    def _drain():
        for s in range(NBUF):
            step = nb - NBUF + s
            sl = lax.rem(jnp.int32(step), NBUF)
            pltpu.make_async_copy(
                slabs.at[sl],
                out_hbm.at[pl.ds(step * VBLK, VBLK), :],
                sems.at[sl],
            ).wait()


def kernel(x, W):
    V, H = W.shape
    (B,) = x.shape
    idx = x.astype(jnp.int32)
    emb = _sc_gather(V, H, B)(W, idx)

    num_blocks = V // VBLK
    outT = pl.pallas_call(
        _mm_body,
        grid=(num_blocks,),
        in_specs=[
            pl.BlockSpec((B, H), lambda i: (0, 0)),
            pl.BlockSpec((VBLK, H), lambda i: (i, 0)),
        ],
        out_specs=pl.BlockSpec(memory_space=pltpu.MemorySpace.HBM),
        out_shape=jax.ShapeDtypeStruct((V, B), jnp.float32),
        scratch_shapes=[
            pltpu.VMEM((NBUF, VBLK, B), jnp.float32),
            pltpu.SemaphoreType.DMA((NBUF,)),
        ],
    )(emb, W)
    return outT.T
